# baseline (speedup 1.0000x reference)
import functools

import jax
import jax.numpy as jnp
from jax import lax
from jax.experimental import pallas as pl
from jax.experimental.pallas import tpu as pltpu
from jax.experimental.pallas import tpu_sc as plsc

VOCAB = 1000000
EMBED_DIM = 64
MAX_SEQ = 2048
BATCH = 4
SEQ = 2048

ROWS = BATCH * SEQ
NT_TC = ROWS // 2
NT_SC = ROWS - NT_TC

NUM_CORES = 2
NUM_SUBCORES = 16
NUM_WORKERS = NUM_CORES * NUM_SUBCORES
ROWS_PER_W = NT_SC // NUM_WORKERS
LANES = 16
COL_CHUNKS = EMBED_DIM // LANES
GROUPS = ROWS_PER_W // LANES
NSLOTS = 8

TC_STEP = 16
TC_GRID = NT_TC // TC_STEP


def _make_sc_kernel():
    mesh = plsc.VectorSubcoreMesh(core_axis_name="c", subcore_axis_name="s")

    @functools.partial(
        pl.kernel,
        mesh=mesh,
        compiler_params=pltpu.CompilerParams(needs_layout_passes=False),
        out_type=jax.ShapeDtypeStruct((NT_SC, EMBED_DIM), jnp.float32),
        scratch_types=[
            pltpu.VMEM((ROWS_PER_W + LANES,), jnp.int32),
            pltpu.VMEM((NSLOTS * EMBED_DIM, 128), jnp.float32),
            pltpu.VMEM((2 * LANES, EMBED_DIM), jnp.float32),
            pltpu.VMEM((ROWS_PER_W, EMBED_DIM), jnp.float32),
            pltpu.SemaphoreType.DMA((NSLOTS,)),
            pltpu.SemaphoreType.DMA,
        ],
    )
    def emb_kernel(x0_hbm, tt_hbm, pos_hbm, out_hbm,
                   idx_v, slab_v, stage_v, pos_v, slab_sems, out_sem):
        wid = lax.axis_index("s") * NUM_CORES + lax.axis_index("c")
        base = wid * ROWS_PER_W
        t0 = (wid % (SEQ // ROWS_PER_W)) * ROWS_PER_W
        iota = lax.iota(jnp.int32, LANES)
        zeros16i = jnp.zeros((LANES,), jnp.int32)

        pltpu.sync_copy(x0_hbm.at[pl.ds(NT_TC + base, ROWS_PER_W)],
                        idx_v.at[pl.ds(0, ROWS_PER_W)])
        idx_v[pl.ds(ROWS_PER_W, LANES)] = zeros16i

        pltpu.sync_copy(pos_hbm.at[pl.ds(t0, ROWS_PER_W)], pos_v)

        @pl.when(t0 == 0)
        def _zero_pos_row0():
            for c in range(COL_CHUNKS):
                pos_v[0, pl.ds(c * LANES, LANES)] = jnp.zeros(
                    (LANES,), jnp.float32
                )

        def fire_slab(tok, slot):
            cstart = pl.multiple_of((tok // 128) * 128, 128)
            pltpu.async_copy(
                tt_hbm.at[:, pl.ds(cstart, 128)],
                slab_v.at[pl.ds(slot * EMBED_DIM, EMBED_DIM)],
                slab_sems.at[slot],
            )

        def wait_slab(slot):
            pltpu.make_async_copy(
                tt_hbm.at[:, pl.ds(0, 128)],
                slab_v.at[pl.ds(slot * EMBED_DIM, EMBED_DIM)],
                slab_sems.at[slot],
            ).wait()

        idxg0 = idx_v[pl.ds(0, LANES)]
        for j in range(NSLOTS):
            fire_slab(idxg0[j], j)

        out_drain = pltpu.make_async_copy(
            stage_v.at[pl.ds(0, LANES)],
            out_hbm.at[pl.ds(base, LANES)],
            out_sem,
        )

        def group_body(g, _):
            idxg = idx_v[pl.ds(g * LANES, LANES)]
            idxh = idx_v[pl.ds(g * LANES + LANES, LANES)]
            maskv = jnp.where(idxg == 0, 0.0, 1.0).astype(jnp.float32)
            p = (g % 2) * LANES

            @pl.when(g >= 2)
            def _wait_out():
                out_drain.wait()

            for j in range(LANES):
                slot = j % NSLOTS
                wait_slab(slot)
                i = idxg[j]
                lvec = (i % 128) + zeros16i
                maskf = maskv[j]
                for kc in range(COL_CHUNKS):
                    rowvec = slot * EMBED_DIM + kc * LANES + iota
                    vals = plsc.load_gather(slab_v, [rowvec, lvec])
                    stage_v[p + j, pl.ds(kc * LANES, LANES)] = (
                        vals * maskf + pos_v[g * LANES + j, pl.ds(kc * LANES, LANES)]
                    )
                nid = idxg[j + NSLOTS] if j < NSLOTS else idxh[j - NSLOTS]
                fire_slab(nid, slot)

            pltpu.async_copy(
                stage_v.at[pl.ds(p, LANES)],
                out_hbm.at[pl.ds(base + g * LANES, LANES)],
                out_sem,
            )
            return 0

        lax.fori_loop(0, GROUPS, group_body, 0)

        for s in range(NSLOTS):
            wait_slab(s)
        out_drain.wait()
        out_drain.wait()

    return emb_kernel


def _tc_body(s_ref, *refs):
    slabs = refs[:TC_STEP]
    pos_ref = refs[TC_STEP]
    out_ref = refs[TC_STEP + 1]
    i = pl.program_id(0)
    lane = lax.broadcasted_iota(jnp.int32, (1, 128), 1)
    for j in range(TC_STEP):
        tok = s_ref[i * TC_STEP + j]
        l = tok % 128
        slab = slabs[j][...]
        onehot = jnp.where(lane == l, 1.0, 0.0).astype(jnp.float32)
        val = lax.dot_general(
            onehot, slab, (((1,), (1,)), ((), ())),
            precision=lax.Precision.HIGHEST,
            preferred_element_type=jnp.float32,
        )
        maskf = jnp.where(tok == 0, 0.0, 1.0).astype(jnp.float32)
        t = (i % (SEQ // TC_STEP)) * TC_STEP + j
        pm = jnp.where(t == 0, 0.0, 1.0).astype(jnp.float32)
        prow = pos_ref[pl.ds(j, 1), :]
        out_ref[pl.ds(j, 1), :] = val * maskf + prow * pm


def _make_tc_kernel():
    def slab_spec(j):
        return pl.BlockSpec(
            (EMBED_DIM, 128),
            lambda i, s, j=j: (0, s[i * TC_STEP + j] // 128),
        )

    return pl.pallas_call(
        _tc_body,
        grid_spec=pltpu.PrefetchScalarGridSpec(
            num_scalar_prefetch=1,
            grid=(TC_GRID,),
            in_specs=[slab_spec(j) for j in range(TC_STEP)]
            + [
                pl.BlockSpec(
                    (TC_STEP, EMBED_DIM),
                    lambda i, s: (i % (SEQ // TC_STEP), 0),
                )
            ],
            out_specs=pl.BlockSpec((TC_STEP, EMBED_DIM), lambda i, s: (i, 0)),
        ),
        out_shape=jax.ShapeDtypeStruct((NT_TC, EMBED_DIM), jnp.float32),
    )


_sc_kernel = _make_sc_kernel()
_tc_kernel = _make_tc_kernel()


@jax.jit
def kernel(x0, x1, token_table, pos_table):
    x0_flat = x0.reshape(ROWS)
    tt = token_table.T
    out_sc = _sc_kernel(x0_flat, tt, pos_table)
    out_tc = _tc_kernel(x0_flat[:NT_TC], *([tt] * TC_STEP), pos_table)
    out = jnp.concatenate(
        [
            out_tc.reshape(NT_TC // SEQ, SEQ, EMBED_DIM),
            out_sc.reshape(NT_SC // SEQ, SEQ, EMBED_DIM),
        ],
        axis=0,
    )
    return out, x1


# speedup vs baseline: 1.2252x; 1.2252x over previous
import functools

import jax
import jax.numpy as jnp
from jax import lax
from jax.experimental import pallas as pl
from jax.experimental.pallas import tpu as pltpu
from jax.experimental.pallas import tpu_sc as plsc

VOCAB = 1000000
EMBED_DIM = 64
MAX_SEQ = 2048
BATCH = 4
SEQ = 2048

ROWS = BATCH * SEQ
NT_TC = ROWS // 2
NT_SC = ROWS - NT_TC

NUM_CORES = 2
NUM_SUBCORES = 16
NUM_WORKERS = NUM_CORES * NUM_SUBCORES
ROWS_PER_W = NT_SC // NUM_WORKERS
LANES = 16
COL_CHUNKS = EMBED_DIM // LANES
GROUPS = ROWS_PER_W // LANES
NSLOTS = 8

TC_STEP = 16
TC_GRID = NT_TC // TC_STEP


def _make_sc_kernel():
    mesh = plsc.VectorSubcoreMesh(core_axis_name="c", subcore_axis_name="s")

    @functools.partial(
        pl.kernel,
        mesh=mesh,
        compiler_params=pltpu.CompilerParams(needs_layout_passes=False),
        out_type=jax.ShapeDtypeStruct((NT_SC, EMBED_DIM), jnp.float32),
        scratch_types=[
            pltpu.VMEM((ROWS_PER_W + LANES,), jnp.int32),
            pltpu.VMEM((NSLOTS * EMBED_DIM, 128), jnp.float32),
            pltpu.VMEM((2 * LANES, EMBED_DIM), jnp.float32),
            pltpu.VMEM((ROWS_PER_W, EMBED_DIM), jnp.float32),
            pltpu.SemaphoreType.DMA((NSLOTS,)),
            pltpu.SemaphoreType.DMA,
        ],
    )
    def emb_kernel(x0_hbm, tt_hbm, pos_hbm, out_hbm,
                   idx_v, slab_v, stage_v, pos_v, slab_sems, out_sem):
        wid = lax.axis_index("s") * NUM_CORES + lax.axis_index("c")
        base = wid * ROWS_PER_W
        t0 = (wid % (SEQ // ROWS_PER_W)) * ROWS_PER_W
        iota = lax.iota(jnp.int32, LANES)
        zeros16i = jnp.zeros((LANES,), jnp.int32)

        pltpu.sync_copy(x0_hbm.at[pl.ds(NT_TC + base, ROWS_PER_W)],
                        idx_v.at[pl.ds(0, ROWS_PER_W)])
        idx_v[pl.ds(ROWS_PER_W, LANES)] = zeros16i

        pltpu.sync_copy(pos_hbm.at[pl.ds(t0, ROWS_PER_W)], pos_v)

        @pl.when(t0 == 0)
        def _zero_pos_row0():
            for c in range(COL_CHUNKS):
                pos_v[0, pl.ds(c * LANES, LANES)] = jnp.zeros(
                    (LANES,), jnp.float32
                )

        def fire_slab(tok, slot):
            cstart = pl.multiple_of((tok // 128) * 128, 128)
            pltpu.async_copy(
                tt_hbm.at[:, pl.ds(cstart, 128)],
                slab_v.at[pl.ds(slot * EMBED_DIM, EMBED_DIM)],
                slab_sems.at[slot],
            )

        def wait_slab(slot):
            pltpu.make_async_copy(
                tt_hbm.at[:, pl.ds(0, 128)],
                slab_v.at[pl.ds(slot * EMBED_DIM, EMBED_DIM)],
                slab_sems.at[slot],
            ).wait()

        idxg0 = idx_v[pl.ds(0, LANES)]
        for j in range(NSLOTS):
            fire_slab(idxg0[j], j)

        out_drain = pltpu.make_async_copy(
            stage_v.at[pl.ds(0, LANES)],
            out_hbm.at[pl.ds(base, LANES)],
            out_sem,
        )

        def group_body(g, _):
            idxg = idx_v[pl.ds(g * LANES, LANES)]
            idxh = idx_v[pl.ds(g * LANES + LANES, LANES)]
            maskv = jnp.where(idxg == 0, 0.0, 1.0).astype(jnp.float32)
            p = (g % 2) * LANES

            @pl.when(g >= 2)
            def _wait_out():
                out_drain.wait()

            for j in range(LANES):
                slot = j % NSLOTS
                wait_slab(slot)
                i = idxg[j]
                lvec = (i % 128) + zeros16i
                maskf = maskv[j]
                for kc in range(COL_CHUNKS):
                    rowvec = slot * EMBED_DIM + kc * LANES + iota
                    vals = plsc.load_gather(slab_v, [rowvec, lvec])
                    stage_v[p + j, pl.ds(kc * LANES, LANES)] = (
                        vals * maskf + pos_v[g * LANES + j, pl.ds(kc * LANES, LANES)]
                    )
                nid = idxg[j + NSLOTS] if j < NSLOTS else idxh[j - NSLOTS]
                fire_slab(nid, slot)

            pltpu.async_copy(
                stage_v.at[pl.ds(p, LANES)],
                out_hbm.at[pl.ds(base + g * LANES, LANES)],
                out_sem,
            )
            return 0

        lax.fori_loop(0, GROUPS, group_body, 0)

        for s in range(NSLOTS):
            wait_slab(s)
        out_drain.wait()
        out_drain.wait()

    return emb_kernel


def _tc_body(s_ref, *refs):
    slabs = refs[:TC_STEP]
    pos_ref = refs[TC_STEP]
    out_ref = refs[TC_STEP + 1]
    i = pl.program_id(0)
    big = jnp.concatenate([slabs[j][...] for j in range(TC_STEP)], axis=1)
    toks = jnp.stack([s_ref[i * TC_STEP + j] for j in range(TC_STEP)])
    lvec = (toks % 128 + 128 * lax.iota(jnp.int32, TC_STEP)).reshape(
        TC_STEP, 1
    )
    lane = lax.broadcasted_iota(jnp.int32, (TC_STEP, 128 * TC_STEP), 1)
    onehot = jnp.where(lane == lvec, 1.0, 0.0).astype(jnp.float32)
    vals = lax.dot_general(
        onehot, big, (((1,), (1,)), ((), ())),
        precision=lax.Precision.HIGHEST,
        preferred_element_type=jnp.float32,
    )
    maskv = jnp.where(toks == 0, 0.0, 1.0).astype(jnp.float32).reshape(
        TC_STEP, 1
    )
    tvec = (i % (SEQ // TC_STEP)) * TC_STEP + lax.broadcasted_iota(
        jnp.int32, (TC_STEP, 1), 0
    )
    pmv = jnp.where(tvec == 0, 0.0, 1.0).astype(jnp.float32)
    out_ref[...] = vals * maskv + pos_ref[...] * pmv


def _make_tc_kernel():
    def slab_spec(j):
        return pl.BlockSpec(
            (EMBED_DIM, 128),
            lambda i, s, j=j: (0, s[i * TC_STEP + j] // 128),
        )

    return pl.pallas_call(
        _tc_body,
        grid_spec=pltpu.PrefetchScalarGridSpec(
            num_scalar_prefetch=1,
            grid=(TC_GRID,),
            in_specs=[slab_spec(j) for j in range(TC_STEP)]
            + [
                pl.BlockSpec(
                    (TC_STEP, EMBED_DIM),
                    lambda i, s: (i % (SEQ // TC_STEP), 0),
                )
            ],
            out_specs=pl.BlockSpec((TC_STEP, EMBED_DIM), lambda i, s: (i, 0)),
        ),
        out_shape=jax.ShapeDtypeStruct((NT_TC, EMBED_DIM), jnp.float32),
    )


_sc_kernel = _make_sc_kernel()
_tc_kernel = _make_tc_kernel()


@jax.jit
def kernel(x0, x1, token_table, pos_table):
    x0_flat = x0.reshape(ROWS)
    tt = token_table.T
    out_sc = _sc_kernel(x0_flat, tt, pos_table)
    out_tc = _tc_kernel(x0_flat[:NT_TC], *([tt] * TC_STEP), pos_table)
    out = jnp.concatenate(
        [
            out_tc.reshape(NT_TC // SEQ, SEQ, EMBED_DIM),
            out_sc.reshape(NT_SC // SEQ, SEQ, EMBED_DIM),
        ],
        axis=0,
    )
    return out, x1


# speedup vs baseline: 2.7611x; 2.2536x over previous
import functools

import jax
import jax.numpy as jnp
from jax import lax
from jax.experimental import pallas as pl
from jax.experimental.pallas import tpu as pltpu
from jax.experimental.pallas import tpu_sc as plsc

VOCAB = 1000000
EMBED_DIM = 64
MAX_SEQ = 2048
BATCH = 4
SEQ = 2048

NUM_CORES = 2
NUM_SUBCORES = 16
NUM_WORKERS = NUM_CORES * NUM_SUBCORES
ROWS = BATCH * SEQ
ROWS_PER_W = ROWS // NUM_WORKERS
T_PER_W = SEQ // (NUM_WORKERS // BATCH)
LANES = 16
COL_CHUNKS = EMBED_DIM // LANES
GROUPS = ROWS_PER_W // LANES
NSLOTS = 8


def _make_sc_kernel():
    mesh = plsc.VectorSubcoreMesh(core_axis_name="c", subcore_axis_name="s")

    @functools.partial(
        pl.kernel,
        mesh=mesh,
        compiler_params=pltpu.CompilerParams(needs_layout_passes=False),
        out_type=jax.ShapeDtypeStruct((ROWS, EMBED_DIM), jnp.float32),
        scratch_types=[
            pltpu.VMEM((ROWS_PER_W + LANES,), jnp.int32),
            pltpu.VMEM((NSLOTS * EMBED_DIM, 128), jnp.float32),
            pltpu.VMEM((2 * LANES, EMBED_DIM), jnp.float32),
            pltpu.VMEM((ROWS_PER_W, EMBED_DIM), jnp.float32),
            pltpu.SemaphoreType.DMA((NSLOTS,)),
            pltpu.SemaphoreType.DMA,
        ],
    )
    def emb_kernel(x0_hbm, tt_hbm, pos_hbm, out_hbm,
                   idx_v, slab_v, stage_v, pos_v, slab_sems, out_sem):
        wid = lax.axis_index("s") * NUM_CORES + lax.axis_index("c")
        base = wid * ROWS_PER_W
        t0 = (wid % (NUM_WORKERS // BATCH)) * T_PER_W
        iota = lax.iota(jnp.int32, LANES)
        zeros16i = jnp.zeros((LANES,), jnp.int32)

        pltpu.sync_copy(x0_hbm.at[pl.ds(base, ROWS_PER_W)],
                        idx_v.at[pl.ds(0, ROWS_PER_W)])
        idx_v[pl.ds(ROWS_PER_W, LANES)] = zeros16i

        pltpu.sync_copy(pos_hbm.at[pl.ds(t0, T_PER_W)], pos_v)

        @pl.when(t0 == 0)
        def _zero_pos_row0():
            for c in range(COL_CHUNKS):
                pos_v[0, pl.ds(c * LANES, LANES)] = jnp.zeros(
                    (LANES,), jnp.float32
                )

        def fire_slab(tok, slot):
            cstart = pl.multiple_of((tok // 128) * 128, 128)
            pltpu.async_copy(
                tt_hbm.at[:, pl.ds(cstart, 128)],
                slab_v.at[pl.ds(slot * EMBED_DIM, EMBED_DIM)],
                slab_sems.at[slot],
            )

        def wait_slab(slot):
            pltpu.make_async_copy(
                tt_hbm.at[:, pl.ds(0, 128)],
                slab_v.at[pl.ds(slot * EMBED_DIM, EMBED_DIM)],
                slab_sems.at[slot],
            ).wait()

        idxg0 = idx_v[pl.ds(0, LANES)]
        for j in range(NSLOTS):
            fire_slab(idxg0[j], j)

        out_drain = pltpu.make_async_copy(
            stage_v.at[pl.ds(0, LANES)],
            out_hbm.at[pl.ds(base, LANES)],
            out_sem,
        )

        def group_body(g, _):
            idxg = idx_v[pl.ds(g * LANES, LANES)]
            idxh = idx_v[pl.ds(g * LANES + LANES, LANES)]
            maskv = jnp.where(idxg == 0, 0.0, 1.0).astype(jnp.float32)
            p = (g % 2) * LANES

            @pl.when(g >= 2)
            def _wait_out():
                out_drain.wait()

            for j in range(LANES):
                slot = j % NSLOTS
                wait_slab(slot)
                i = idxg[j]
                lvec = (i % 128) + zeros16i
                maskf = maskv[j]
                for kc in range(COL_CHUNKS):
                    rowvec = slot * EMBED_DIM + kc * LANES + iota
                    vals = plsc.load_gather(slab_v, [rowvec, lvec])
                    stage_v[p + j, pl.ds(kc * LANES, LANES)] = (
                        vals * maskf + pos_v[g * LANES + j, pl.ds(kc * LANES, LANES)]
                    )
                nid = idxg[j + NSLOTS] if j < NSLOTS else idxh[j - NSLOTS]
                fire_slab(nid, slot)

            pltpu.async_copy(
                stage_v.at[pl.ds(p, LANES)],
                out_hbm.at[pl.ds(base + g * LANES, LANES)],
                out_sem,
            )
            return 0

        lax.fori_loop(0, GROUPS, group_body, 0)

        for s in range(NSLOTS):
            wait_slab(s)
        out_drain.wait()
        out_drain.wait()

    return emb_kernel


_sc_kernel = _make_sc_kernel()


@jax.jit
def kernel(x0, x1, token_table, pos_table):
    x0_flat = x0.reshape(ROWS)
    out = _sc_kernel(x0_flat, token_table.T, pos_table)
    return out.reshape(BATCH, SEQ, EMBED_DIM), x1


# speedup vs baseline: 3.1545x; 1.1425x over previous
import functools

import jax
import jax.numpy as jnp
from jax import lax
from jax.experimental import pallas as pl
from jax.experimental.pallas import tpu as pltpu
from jax.experimental.pallas import tpu_sc as plsc

VOCAB = 1000000
EMBED_DIM = 64
MAX_SEQ = 2048
BATCH = 4
SEQ = 2048

NUM_CORES = 2
NUM_SUBCORES = 16
NUM_WORKERS = NUM_CORES * NUM_SUBCORES
ROWS = BATCH * SEQ
ROWS_PER_W = ROWS // NUM_WORKERS
T_PER_W = SEQ // (NUM_WORKERS // BATCH)
LANES = 16
COL_CHUNKS = EMBED_DIM // LANES
GROUPS = ROWS_PER_W // LANES
NSLOTS = 8


def _make_sc_kernel():
    mesh = plsc.VectorSubcoreMesh(core_axis_name="c", subcore_axis_name="s")

    @functools.partial(
        pl.kernel,
        mesh=mesh,
        compiler_params=pltpu.CompilerParams(needs_layout_passes=False),
        out_type=jax.ShapeDtypeStruct((ROWS, EMBED_DIM), jnp.float32),
        scratch_types=[
            pltpu.VMEM((ROWS_PER_W + LANES,), jnp.int32),
            pltpu.VMEM((NSLOTS * EMBED_DIM, 128), jnp.float32),
            pltpu.VMEM((2 * LANES, EMBED_DIM), jnp.float32),
            pltpu.VMEM((ROWS_PER_W, EMBED_DIM), jnp.float32),
            pltpu.SemaphoreType.DMA((NSLOTS,)),
            pltpu.SemaphoreType.DMA,
        ],
    )
    def emb_kernel(x0_hbm, tt_hbm, pos_hbm, out_hbm,
                   idx_v, slab_v, stage_v, pos_v, slab_sems, out_sem):
        wid = lax.axis_index("s") * NUM_CORES + lax.axis_index("c")
        base = wid * ROWS_PER_W
        t0 = (wid % (NUM_WORKERS // BATCH)) * T_PER_W
        iota = lax.iota(jnp.int32, LANES)
        zeros16i = jnp.zeros((LANES,), jnp.int32)

        pltpu.sync_copy(x0_hbm.at[pl.ds(base, ROWS_PER_W)],
                        idx_v.at[pl.ds(0, ROWS_PER_W)])
        idx_v[pl.ds(ROWS_PER_W, LANES)] = zeros16i

        pltpu.sync_copy(pos_hbm.at[pl.ds(t0, T_PER_W)], pos_v)

        @pl.when(t0 == 0)
        def _zero_pos_row0():
            for c in range(COL_CHUNKS):
                pos_v[0, pl.ds(c * LANES, LANES)] = jnp.zeros(
                    (LANES,), jnp.float32
                )

        def fire_slab(tok, slot):
            cstart = pl.multiple_of((tok // 128) * 128, 128)
            pltpu.async_copy(
                tt_hbm.at[:, pl.ds(cstart, 128)],
                slab_v.at[pl.ds(slot * EMBED_DIM, EMBED_DIM)],
                slab_sems.at[slot],
            )

        def wait_slab(slot):
            pltpu.make_async_copy(
                tt_hbm.at[:, pl.ds(0, 128)],
                slab_v.at[pl.ds(slot * EMBED_DIM, EMBED_DIM)],
                slab_sems.at[slot],
            ).wait()

        idxg0 = idx_v[pl.ds(0, LANES)]
        for j in range(NSLOTS):
            fire_slab(idxg0[j], j)

        out_drain = pltpu.make_async_copy(
            stage_v.at[pl.ds(0, LANES)],
            out_hbm.at[pl.ds(base, LANES)],
            out_sem,
        )

        def group_body(g, _):
            idxg = idx_v[pl.ds(g * LANES, LANES)]
            idxh = idx_v[pl.ds(g * LANES + LANES, LANES)]
            maskv = jnp.where(idxg == 0, 0.0, 1.0).astype(jnp.float32)
            p = (g % 2) * LANES

            @pl.when(g >= 2)
            def _wait_out():
                out_drain.wait()

            for j in range(LANES):
                slot = j % NSLOTS
                wait_slab(slot)
                i = idxg[j]
                lvec = (i % 128) + zeros16i
                maskf = maskv[j]
                for kc in range(COL_CHUNKS):
                    rowvec = slot * EMBED_DIM + kc * LANES + iota
                    vals = plsc.load_gather(slab_v, [rowvec, lvec])
                    stage_v[p + j, pl.ds(kc * LANES, LANES)] = (
                        vals * maskf + pos_v[g * LANES + j, pl.ds(kc * LANES, LANES)]
                    )
                if j < NSLOTS:
                    fire_slab(idxg[j + NSLOTS], slot)
                else:
                    nid = idxh[j - NSLOTS]

                    @pl.when(g < GROUPS - 1)
                    def _fire_next():
                        fire_slab(nid, slot)

            pltpu.async_copy(
                stage_v.at[pl.ds(p, LANES)],
                out_hbm.at[pl.ds(base + g * LANES, LANES)],
                out_sem,
            )
            return 0

        lax.fori_loop(0, GROUPS, group_body, 0)

        out_drain.wait()
        out_drain.wait()

    return emb_kernel


_sc_kernel = _make_sc_kernel()


@jax.jit
def kernel(x0, x1, token_table, pos_table):
    x0_flat = x0.reshape(ROWS)
    out = _sc_kernel(x0_flat, token_table.T, pos_table)
    return out.reshape(BATCH, SEQ, EMBED_DIM), x1
